# baseline (speedup 1.0000x reference)
import functools

import jax
import jax.numpy as jnp
from jax import lax
from jax.experimental import pallas as pl
from jax.experimental.pallas import tpu as pltpu
from jax.experimental.pallas import tpu_sc as plsc

N = 10000
E = 160000
D = 256
H = D // 2
NS = 16
RPT = 632
NPAD = NS * RPT
CH = 128
EPT = 10240
EPAD = EPT * NS
NCH = EPT // CH
RD = 8
LA = RD // 2
BLK = 2000

assert (NCH - 2 * LA) % RD == 0


def _sc_body(feat_lo, feat_hi, idx_hbm, zacc, zdeg, ones_hbm,
             out_sum, out_deg, acc, dacc, idx_v, sidx, didx, *rest):
    brows = rest[:RD]
    gsem = rest[RD:2 * RD]
    ssem = rest[2 * RD:3 * RD]
    c = lax.axis_index("c")
    s = lax.axis_index("s")
    r0 = s * RPT

    pltpu.sync_copy(zacc.at[pl.ds(r0, RPT)], acc.at[pl.ds(r0, RPT)])
    pltpu.sync_copy(zdeg.at[pl.ds(r0, RPT)], dacc.at[pl.ds(r0, RPT)])
    pltpu.sync_copy(idx_hbm.at[s], idx_v)
    pltpu.sync_copy(ones_hbm, ones_v := rest[3 * RD])
    plsc.subcore_barrier()

    feat_c = [feat_lo, feat_hi]
    lo_mask = jnp.full((16,), 0xFFFF, jnp.int32)

    def gather(k, p):
        for g in range(CH // 16):
            w = idx_v[k, pl.ds(g * 16, 16)]
            sidx[p, pl.ds(g * 16, 16)] = lax.bitwise_and(w, lo_mask)

        @pl.when(c == 0)
        def _():
            pltpu.async_copy(feat_c[0].at[sidx.at[p]], brows[p], gsem[p])

        @pl.when(c == 1)
        def _():
            pltpu.async_copy(feat_c[1].at[sidx.at[p]], brows[p], gsem[p])

    def gwait(p):
        pltpu.make_async_copy(feat_c[0].at[sidx.at[p]], brows[p],
                              gsem[p]).wait()

    def scatter(k, p):
        for g in range(CH // 16):
            w = idx_v[k, pl.ds(g * 16, 16)]
            didx[p, pl.ds(g * 16, 16)] = lax.shift_right_logical(w, 16)
        pltpu.async_copy(brows[p], acc.at[didx.at[p]], ssem[p], add=True)
        pltpu.async_copy(ones_v, dacc.at[didx.at[p]], ssem[p], add=True)

    def swait(p):
        pltpu.make_async_copy(brows[p], acc.at[didx.at[p]], ssem[p]).wait()
        pltpu.make_async_copy(ones_v, dacc.at[didx.at[p]], ssem[p]).wait()

    for p in range(LA):
        gather(p, p)
    for k in range(LA):
        gwait(k)
        scatter(k, k)
        gather(k + LA, k + LA)

    def body(i, carry):
        base = LA + RD * i
        for j in range(RD):
            k = base + j
            p = (LA + j) % RD
            q = (p + LA) % RD
            gwait(p)
            scatter(k, p)
            swait(q)
            gather(k + LA, q)
        return carry

    lax.fori_loop(0, (NCH - 2 * LA) // RD, body, 0)

    for j in range(LA):
        p = RD - LA + j
        gwait(p)
        scatter(NCH - LA + j, p)
    for p in range(RD):
        swait(p)
    plsc.subcore_barrier()

    pltpu.sync_copy(acc.at[pl.ds(r0, RPT)], out_sum.at[c, pl.ds(r0, RPT)])
    pltpu.sync_copy(dacc.at[pl.ds(r0, RPT)], out_deg.at[c, pl.ds(r0, RPT)])


_sc_fn = pl.kernel(
    _sc_body,
    out_type=[
        jax.ShapeDtypeStruct((2, NPAD, H), jnp.bfloat16),
        jax.ShapeDtypeStruct((2, NPAD, 16), jnp.float32),
    ],
    mesh=plsc.VectorSubcoreMesh(core_axis_name="c", subcore_axis_name="s"),
    scratch_types=[
        pltpu.VMEM_SHARED((NPAD, H), jnp.bfloat16),
        pltpu.VMEM_SHARED((NPAD, 16), jnp.float32),
        pltpu.VMEM((NCH, CH), jnp.int32),
        pltpu.VMEM((RD, CH), jnp.int32),
        pltpu.VMEM((RD, CH), jnp.int32),
    ] + [pltpu.VMEM((CH, H), jnp.bfloat16) for _ in range(RD)]
      + [pltpu.SemaphoreType.DMA for _ in range(2 * RD)]
      + [pltpu.VMEM((CH, 16), jnp.float32)],
    compiler_params=pltpu.CompilerParams(use_tc_tiling_on_sc=False,
                                         needs_layout_passes=False),
)


def _tc_body(feat_ref, slo_ref, shi_ref, d0_ref, d1_ref,
             wst_ref, wnl_ref, wnh_ref, out_ref):
    deg = d0_ref[:, 0:1] + d1_ref[:, 0:1]
    r = 1.0 / jnp.maximum(deg, 1.0)
    acc = jnp.dot(feat_ref[...], wst_ref[...],
                  preferred_element_type=jnp.float32)
    acc = acc + jnp.dot(slo_ref[...].astype(jnp.float32) * r, wnl_ref[...],
                        preferred_element_type=jnp.float32)
    acc = acc + jnp.dot(shi_ref[...].astype(jnp.float32) * r, wnh_ref[...],
                        preferred_element_type=jnp.float32)
    out_ref[...] = acc


_tc_fn = pl.pallas_call(
    _tc_body,
    grid=(N // BLK,),
    in_specs=[
        pl.BlockSpec((BLK, D), lambda i: (i, 0)),
        pl.BlockSpec((BLK, H), lambda i: (i, 0)),
        pl.BlockSpec((BLK, H), lambda i: (i, 0)),
        pl.BlockSpec((BLK, 16), lambda i: (i, 0)),
        pl.BlockSpec((BLK, 16), lambda i: (i, 0)),
        pl.BlockSpec((D, D), lambda i: (0, 0)),
        pl.BlockSpec((H, D), lambda i: (0, 0)),
        pl.BlockSpec((H, D), lambda i: (0, 0)),
    ],
    out_specs=pl.BlockSpec((BLK, D), lambda i: (i, 0)),
    out_shape=jax.ShapeDtypeStruct((N, D), jnp.float32),
)


def kernel(feat, edge_index, W_self, W_neigh):
    src = edge_index[0].astype(jnp.int32)
    dst = edge_index[1].astype(jnp.int32)
    pad = EPAD - E
    src_p = jnp.concatenate([src, jnp.zeros((pad,), jnp.int32)])
    dst_p = jnp.concatenate([dst, jnp.full((pad,), N + 8, jnp.int32)])
    idx_p = (src_p | (dst_p << 16)).reshape(NS, NCH, CH)
    feat_b = feat.astype(jnp.bfloat16)
    feat_lo = feat_b[:, :H]
    feat_hi = feat_b[:, H:]
    zacc = jnp.zeros((NPAD, H), jnp.bfloat16)
    zdeg = jnp.zeros((NPAD, 16), jnp.float32)
    ones = jnp.full((CH, 16), 0.5, jnp.float32)

    sums, degs = _sc_fn(feat_lo, feat_hi, idx_p, zacc, zdeg, ones)

    return _tc_fn(feat, sums[0], sums[1], degs[0], degs[1],
                  W_self.T, W_neigh.T[:H], W_neigh.T[H:])
